# baseline (speedup 1.0000x reference)
import functools

import jax
import jax.numpy as jnp
from jax import lax
from jax.experimental import pallas as pl
from jax.experimental.pallas import tpu as pltpu
from jax.experimental.pallas import tpu_sc as plsc

N = 10000
D = 128
E = 320000
PROJ = 64

NC = 2
NS = 16
CHUNK = 128
CH_T = 160
HCH = 40
NPART = CH_T // HCH
E_T = CH_T * CHUNK
E_PAD = NS * E_T
N_ACC = 10240
ZROWS = 128
ZREP = N_ACC // (NS * ZROWS)
WB = 632
WB_LAST = N - WB * (NS - 1)

_HIGH = lax.Precision.HIGHEST


def _mm(x, w):
    return jnp.dot(x, w, precision=_HIGH, preferred_element_type=jnp.float32)


def _elu(x):
    return jnp.where(x > 0, x, jnp.exp(x) - 1.0)


def _segsum_body(h_hbm, src_hbm, dst_hbm, out_hbm,
                 src_all, dst_all, rows0_v, rows1_v, acc_sh, sem0, sem1):
    cid = lax.axis_index("c")
    sid = lax.axis_index("s")

    zero16 = jnp.zeros((16,), jnp.float32)

    def _zrow(i, _):
        def _zcol(j, _):
            rows0_v[i, pl.ds(j * 16, 16)] = zero16
            return 0
        return lax.fori_loop(0, D // 16, _zcol, 0)

    lax.fori_loop(0, ZROWS, _zrow, 0)
    for k in range(ZREP):
        pltpu.sync_copy(rows0_v.at[pl.ds(0, ZROWS)],
                        acc_sh.at[pl.ds((sid * ZREP + k) * ZROWS, ZROWS)])
    plsc.subcore_barrier()

    bufs = ((rows0_v, sem0), (rows1_v, sem1))

    for enc in range(NC):
        @pl.when(cid == enc)
        def _():
            h_enc = h_hbm.at[enc]
            for part in range(NPART):
                pltpu.sync_copy(
                    src_hbm.at[enc, sid, pl.ds(part * HCH, HCH)], src_all)
                pltpu.sync_copy(
                    dst_hbm.at[enc, sid, pl.ds(part * HCH, HCH)], dst_all)
                for b, (rows, sem) in enumerate(bufs):
                    pltpu.async_copy(h_enc.at[src_all.at[b]], rows, sem)

                def _outer(g, _):
                    for b, (rows, sem) in enumerate(bufs):
                        j = g * 2 + b
                        pltpu.make_async_copy(h_enc.at[src_all.at[j]],
                                              rows, sem).wait()
                        pltpu.sync_copy(rows, acc_sh.at[dst_all.at[j]],
                                        add=True)

                        @pl.when(j + 2 < HCH)
                        def _():
                            pltpu.async_copy(h_enc.at[src_all.at[j + 2]],
                                             rows, sem)
                    return 0

                lax.fori_loop(0, HCH // 2, _outer, 0)
    plsc.subcore_barrier()

    @pl.when(sid < NS - 1)
    def _():
        pltpu.sync_copy(acc_sh.at[pl.ds(sid * WB, WB)],
                        out_hbm.at[cid, pl.ds(sid * WB, WB)])

    @pl.when(sid == NS - 1)
    def _():
        pltpu.sync_copy(acc_sh.at[pl.ds((NS - 1) * WB, WB_LAST)],
                        out_hbm.at[cid, pl.ds((NS - 1) * WB, WB_LAST)])


_segsum = pl.kernel(
    _segsum_body,
    out_type=jax.ShapeDtypeStruct((NC, N, D), jnp.float32),
    mesh=plsc.VectorSubcoreMesh(core_axis_name="c", subcore_axis_name="s"),
    scratch_types=[
        pltpu.VMEM((HCH, CHUNK), jnp.int32),
        pltpu.VMEM((HCH, CHUNK), jnp.int32),
        pltpu.VMEM((CHUNK, D), jnp.float32),
        pltpu.VMEM((CHUNK, D), jnp.float32),
        pltpu.VMEM_SHARED((N_ACC, D), jnp.float32),
        pltpu.SemaphoreType.DMA,
        pltpu.SemaphoreType.DMA,
    ],
)


def _pad_edges(ei):
    pad = E_PAD - E
    r = jnp.arange(pad, dtype=jnp.int32)
    pad_src = (r * 97) % N
    pad_dst = N + r % (N_ACC - N)
    src = jnp.concatenate([ei[0], pad_src]).reshape(NS, CH_T, CHUNK)
    dst = jnp.concatenate([ei[1], pad_dst]).reshape(NS, CH_T, CHUNK)
    return src, dst


NB = 10
BLK = N // NB

_h_spec = pl.BlockSpec((1, BLK, D), lambda e, i: (e, i, 0))
_w_spec = pl.BlockSpec((1, D, D), lambda e, i: (e, 0, 0))
_b_spec = pl.BlockSpec((1, 1, D), lambda e, i: (e, 0, 0))


def _linear_body(x_ref, w_ref, b_ref, o_ref):
    o_ref[0] = _mm(x_ref[0], w_ref[0]) + b_ref[0]


_linear_call = pl.pallas_call(
    _linear_body,
    grid=(2, NB),
    in_specs=[_h_spec, _w_spec, _b_spec],
    out_specs=_h_spec,
    out_shape=jax.ShapeDtypeStruct((2, N, D), jnp.float32),
)


def _gin_body(scal_ref, h_ref, a_ref, w1_ref, b1_ref, w2_ref, b2_ref,
              g_ref, be_ref, o_ref):
    e = pl.program_id(0)
    h = h_ref[0]
    z = scal_ref[e] * h + a_ref[0]
    t = _elu(_mm(z, w1_ref[0]) + b1_ref[0])
    t = _elu(_mm(t, w2_ref[0]) + b2_ref[0])
    y = h + t
    mu = jnp.mean(y, axis=-1, keepdims=True)
    yc = y - mu
    var = jnp.mean(yc * yc, axis=-1, keepdims=True)
    o_ref[0] = yc * lax.rsqrt(var + 1e-5) * g_ref[0] + be_ref[0]


_gin_call = pl.pallas_call(
    _gin_body,
    grid=(2, NB),
    in_specs=[
        pl.BlockSpec(memory_space=pltpu.SMEM),
        _h_spec, _h_spec,
        _w_spec, _b_spec, _w_spec, _b_spec, _b_spec, _b_spec,
    ],
    out_specs=_h_spec,
    out_shape=jax.ShapeDtypeStruct((2, N, D), jnp.float32),
)


def _node_body(h_ref, wn_ref, bn_ref, w1_ref, b1_ref, w2_ref, b2_ref,
               o_ref, g_ref):
    i = pl.program_id(1)
    h = h_ref[0]
    ne = _mm(h, wn_ref[0]) + bn_ref[0]
    t = _elu(_mm(ne, w1_ref[0]) + b1_ref[0])
    y = _mm(t, w2_ref[0]) + b2_ref[0]
    nrm = jnp.sqrt(jnp.sum(y * y, axis=-1, keepdims=True))
    o_ref[0] = y / jnp.maximum(nrm, 1e-12)

    s = jnp.sum(h, axis=0, keepdims=True)

    @pl.when(i == 0)
    def _():
        g_ref[0] = s

    @pl.when(i > 0)
    def _():
        g_ref[0] += s


_node_call = pl.pallas_call(
    _node_body,
    grid=(2, NB),
    in_specs=[
        _h_spec, _w_spec, _b_spec,
        _w_spec, _b_spec,
        pl.BlockSpec((1, D, PROJ), lambda e, i: (e, 0, 0)),
        pl.BlockSpec((1, 1, PROJ), lambda e, i: (e, 0, 0)),
    ],
    out_specs=[
        pl.BlockSpec((1, BLK, PROJ), lambda e, i: (e, i, 0)),
        pl.BlockSpec((1, 1, D), lambda e, i: (e, 0, 0)),
    ],
    out_shape=[
        jax.ShapeDtypeStruct((2, N, PROJ), jnp.float32),
        jax.ShapeDtypeStruct((2, 1, D), jnp.float32),
    ],
)


def _graph_body(gs_ref, g1w_ref, g1b_ref, g2w_ref, g2b_ref,
                l1w_ref, l1b_ref, l2w_ref, l2b_ref, o_ref):
    for e in range(2):
        g = gs_ref[e:e + 1, :] * (1.0 / N)
        t = _elu(_mm(g, g1w_ref[e]) + g1b_ref[e:e + 1, :])
        ge = _mm(t, g2w_ref[e]) + g2b_ref[e:e + 1, :]
        t = _elu(_mm(ge, l1w_ref[e]) + l1b_ref[e:e + 1, :])
        y = _mm(t, l2w_ref[e]) + l2b_ref[e:e + 1, :]
        nrm = jnp.sqrt(jnp.sum(y * y, axis=-1, keepdims=True))
        o_ref[e:e + 1, :] = y / jnp.maximum(nrm, 1e-12)


_graph_call = pl.pallas_call(
    _graph_body,
    out_shape=jax.ShapeDtypeStruct((2, PROJ), jnp.float32),
)


def _stack_lin(pa, pb, last=D):
    return (jnp.stack([pa["W"], pb["W"]]),
            jnp.stack([pa["b"].reshape(1, last), pb["b"].reshape(1, last)]))


def kernel(sc_x, fc_x, params, sc_edge_index, fc_edge_index):
    enc = (params["sc_enc"], params["fc_enc"])
    src_sc, dst_sc = _pad_edges(sc_edge_index)
    src_fc, dst_fc = _pad_edges(fc_edge_index)
    src = jnp.stack([src_sc, src_fc])
    dst = jnp.stack([dst_sc, dst_fc])

    x = jnp.stack([sc_x, fc_x])
    w, b = _stack_lin(enc[0]["input_proj"], enc[1]["input_proj"])
    h = _linear_call(x, w, b)

    for li in range(3):
        lp = (enc[0]["layers"][li], enc[1]["layers"][li])
        agg = _segsum(h, src, dst)
        scal = jnp.stack([1.0 + lp[0]["eps"], 1.0 + lp[1]["eps"]])
        w1, b1 = _stack_lin(lp[0]["mlp1"], lp[1]["mlp1"])
        w2, b2 = _stack_lin(lp[0]["mlp2"], lp[1]["mlp2"])
        g = jnp.stack([lp[0]["ln_g"].reshape(1, D), lp[1]["ln_g"].reshape(1, D)])
        be = jnp.stack([lp[0]["ln_b"].reshape(1, D), lp[1]["ln_b"].reshape(1, D)])
        h = _gin_call(scal, h, agg, w1, b1, w2, b2, g, be)

    nh = (params["sc_node_proj"], params["fc_node_proj"])
    wn, bn = _stack_lin(enc[0]["node_proj"], enc[1]["node_proj"])
    w1, b1 = _stack_lin(nh[0]["l1"], nh[1]["l1"])
    w2, b2 = _stack_lin(nh[0]["l2"], nh[1]["l2"], last=PROJ)
    z_node, gsum = _node_call(h, wn, bn, w1, b1, w2, b2)

    pj = (params["sc_proj"], params["fc_proj"])
    zg = _graph_call(
        gsum.reshape(2, D),
        jnp.stack([enc[0]["graph_proj1"]["W"], enc[1]["graph_proj1"]["W"]]),
        jnp.stack([enc[0]["graph_proj1"]["b"], enc[1]["graph_proj1"]["b"]]),
        jnp.stack([enc[0]["graph_proj2"]["W"], enc[1]["graph_proj2"]["W"]]),
        jnp.stack([enc[0]["graph_proj2"]["b"], enc[1]["graph_proj2"]["b"]]),
        jnp.stack([pj[0]["l1"]["W"], pj[1]["l1"]["W"]]),
        jnp.stack([pj[0]["l1"]["b"], pj[1]["l1"]["b"]]),
        jnp.stack([pj[0]["l2"]["W"], pj[1]["l2"]["W"]]),
        jnp.stack([pj[0]["l2"]["b"], pj[1]["l2"]["b"]]),
    )
    return (zg[0:1], zg[1:2], z_node[0], z_node[1])


# speedup vs baseline: 1.3481x; 1.3481x over previous
import functools

import jax
import jax.numpy as jnp
from jax import lax
from jax.experimental import pallas as pl
from jax.experimental.pallas import tpu as pltpu
from jax.experimental.pallas import tpu_sc as plsc

N = 10000
D = 128
E = 320000
PROJ = 64

NC = 2
NS = 16
NW = NC * NS
CHUNK = 96
CH_PER_W = 108
HCH = 36
NPART = CH_PER_W // HCH
E_W = CH_PER_W * CHUNK
E_PAD = NW * E_W
N_ACC = 10240
ZROWS = 64
ZREP = N_ACC // (NS * ZROWS)
WB = 632
WB_LAST = N - WB * (NS - 1)

_HIGH = lax.Precision.HIGHEST


def _mm(x, w):
    return jnp.dot(x, w, precision=lax.Precision.DEFAULT, preferred_element_type=jnp.float32)


def _elu(x):
    return jnp.where(x > 0, x, jnp.exp(x) - 1.0)


def _segsum_body(h_hbm, src_hbm, dst_hbm, out_hbm,
                 src_all, dst_all, rows0_v, rows1_v, rows2_v, acc_sh,
                 gsem0, gsem1, gsem2, ssem0, ssem1, ssem2):
    cid = lax.axis_index("c")
    sid = lax.axis_index("s")
    wid = sid * NC + cid

    zero16 = jnp.zeros((16,), jnp.float32)

    def _zrow(i, _):
        def _zcol(j, _):
            rows0_v[i, pl.ds(j * 16, 16)] = zero16
            return 0
        return lax.fori_loop(0, D // 16, _zcol, 0)

    lax.fori_loop(0, ZROWS, _zrow, 0)

    for k in range(ZREP):
        pltpu.sync_copy(rows0_v.at[pl.ds(0, ZROWS)],
                        acc_sh.at[pl.ds((sid * ZREP + k) * ZROWS, ZROWS)])
    plsc.subcore_barrier()

    bufs = ((rows0_v, gsem0, ssem0), (rows1_v, gsem1, ssem1),
            (rows2_v, gsem2, ssem2))

    def _gather(j_row, rows, gsem):
        pltpu.async_copy(h_hbm.at[src_all.at[j_row]], rows, gsem)

    def _wait_gather(j_row, rows, gsem):
        pltpu.make_async_copy(h_hbm.at[src_all.at[j_row]], rows, gsem).wait()

    def _scatter(j_row, rows, ssem):
        pltpu.async_copy(rows, acc_sh.at[dst_all.at[j_row]], ssem, add=True)

    def _wait_scatter(rows, ssem):
        pltpu.make_async_copy(rows, acc_sh.at[dst_all.at[0]], ssem).wait()

    for part in range(NPART):
        pltpu.sync_copy(src_hbm.at[wid, part], src_all)
        pltpu.sync_copy(dst_hbm.at[wid, part], dst_all)
        for b in range(2):
            _gather(b, bufs[b][0], bufs[b][1])

        def _ring(g, _):
            for b in range(3):
                j = g * 3 + b
                rows, gsem, ssem = bufs[b]
                rows_p, gsem_p, ssem_p = bufs[(b + 2) % 3]
                _wait_gather(j, rows, gsem)
                _scatter(j, rows, ssem)

                @pl.when(j + 2 < HCH)
                def _():
                    @pl.when(j >= 1)
                    def _():
                        _wait_scatter(rows_p, ssem_p)
                    _gather(j + 2, rows_p, gsem_p)
            return 0

        lax.fori_loop(0, HCH // 3, _ring, 0)
        for b in range(3):
            _wait_scatter(bufs[b][0], bufs[b][2])
    plsc.subcore_barrier()

    @pl.when(sid < NS - 1)
    def _():
        pltpu.sync_copy(acc_sh.at[pl.ds(sid * WB, WB)],
                        out_hbm.at[cid, pl.ds(sid * WB, WB)])

    @pl.when(sid == NS - 1)
    def _():
        pltpu.sync_copy(acc_sh.at[pl.ds((NS - 1) * WB, WB_LAST)],
                        out_hbm.at[cid, pl.ds((NS - 1) * WB, WB_LAST)])


_segsum = pl.kernel(
    _segsum_body,
    out_type=jax.ShapeDtypeStruct((NC, N, D), jnp.float32),
    mesh=plsc.VectorSubcoreMesh(core_axis_name="c", subcore_axis_name="s"),
    scratch_types=[
        pltpu.VMEM((HCH, CHUNK), jnp.int32),
        pltpu.VMEM((HCH, CHUNK), jnp.int32),
        pltpu.VMEM((CHUNK, D), jnp.float32),
        pltpu.VMEM((CHUNK, D), jnp.float32),
        pltpu.VMEM((CHUNK, D), jnp.float32),
        pltpu.VMEM_SHARED((N_ACC, D), jnp.float32),
        pltpu.SemaphoreType.DMA,
        pltpu.SemaphoreType.DMA,
        pltpu.SemaphoreType.DMA,
        pltpu.SemaphoreType.DMA,
        pltpu.SemaphoreType.DMA,
        pltpu.SemaphoreType.DMA,
    ],
)


def _pad_edges(ei):
    pad = E_PAD - E
    r = jnp.arange(pad, dtype=jnp.int32)
    pad_src = (r * 97) % N
    pad_dst = N + r % (N_ACC - N)
    src = jnp.concatenate([ei[0], pad_src]).reshape(NW, NPART, HCH, CHUNK)
    dst = jnp.concatenate([ei[1], pad_dst]).reshape(NW, NPART, HCH, CHUNK)
    return src, dst


NB = 10
BLK = N // NB

_row_spec = pl.BlockSpec((BLK, D), lambda i: (i, 0))
_w_spec = pl.BlockSpec((D, D), lambda i: (0, 0))
_b_spec = pl.BlockSpec((1, D), lambda i: (0, 0))


def _linear_body(x_ref, w_ref, b_ref, o_ref):
    o_ref[...] = _mm(x_ref[...], w_ref[...]) + b_ref[...]


_linear_call = pl.pallas_call(
    _linear_body,
    grid=(NB,),
    in_specs=[_row_spec, _w_spec, _b_spec],
    out_specs=_row_spec,
    out_shape=jax.ShapeDtypeStruct((N, D), jnp.float32),
)


def _linear(x, p):
    return _linear_call(x, p["W"], p["b"].reshape(1, D))


def _gin_body(scal_ref, h_ref, a_ref, w1_ref, b1_ref, w2_ref, b2_ref,
              g_ref, be_ref, o_ref):
    h = h_ref[...]
    z = scal_ref[0] * h + a_ref[0] + a_ref[1]
    t = _elu(_mm(z, w1_ref[...]) + b1_ref[...])
    t = _elu(_mm(t, w2_ref[...]) + b2_ref[...])
    y = h + t
    mu = jnp.mean(y, axis=-1, keepdims=True)
    yc = y - mu
    var = jnp.mean(yc * yc, axis=-1, keepdims=True)
    o_ref[...] = yc * lax.rsqrt(var + 1e-5) * g_ref[...] + be_ref[...]


_gin_call = pl.pallas_call(
    _gin_body,
    grid=(NB,),
    in_specs=[
        pl.BlockSpec(memory_space=pltpu.SMEM),
        _row_spec,
        pl.BlockSpec((NC, BLK, D), lambda i: (0, i, 0)),
        _w_spec, _b_spec, _w_spec, _b_spec, _b_spec, _b_spec,
    ],
    out_specs=_row_spec,
    out_shape=jax.ShapeDtypeStruct((N, D), jnp.float32),
)


def _gin(h, agg2, lp):
    scal = (1.0 + lp["eps"]).reshape(1)
    return _gin_call(scal, h, agg2,
                     lp["mlp1"]["W"], lp["mlp1"]["b"].reshape(1, D),
                     lp["mlp2"]["W"], lp["mlp2"]["b"].reshape(1, D),
                     lp["ln_g"].reshape(1, D), lp["ln_b"].reshape(1, D))


def _node_body(h_ref, wn_ref, bn_ref, w1_ref, b1_ref, w2_ref, b2_ref,
               o_ref, g_ref):
    i = pl.program_id(0)
    h = h_ref[...]
    ne = _mm(h, wn_ref[...]) + bn_ref[...]
    t = _elu(_mm(ne, w1_ref[...]) + b1_ref[...])
    y = _mm(t, w2_ref[...]) + b2_ref[...]
    nrm = jnp.sqrt(jnp.sum(y * y, axis=-1, keepdims=True))
    o_ref[...] = y / jnp.maximum(nrm, 1e-12)

    s = jnp.sum(h, axis=0, keepdims=True)

    @pl.when(i == 0)
    def _():
        g_ref[...] = s

    @pl.when(i > 0)
    def _():
        g_ref[...] += s


_node_call = pl.pallas_call(
    _node_body,
    grid=(NB,),
    in_specs=[
        _row_spec, _w_spec, _b_spec, _w_spec, _b_spec,
        pl.BlockSpec((D, PROJ), lambda i: (0, 0)),
        pl.BlockSpec((1, PROJ), lambda i: (0, 0)),
    ],
    out_specs=[
        pl.BlockSpec((BLK, PROJ), lambda i: (i, 0)),
        pl.BlockSpec((1, D), lambda i: (0, 0)),
    ],
    out_shape=[
        jax.ShapeDtypeStruct((N, PROJ), jnp.float32),
        jax.ShapeDtypeStruct((1, D), jnp.float32),
    ],
)


def _node_head(h, pn, ph):
    return _node_call(h, pn["W"], pn["b"].reshape(1, D),
                      ph["l1"]["W"], ph["l1"]["b"].reshape(1, D),
                      ph["l2"]["W"], ph["l2"]["b"].reshape(1, PROJ))


def _graph_body(gs_ref, g1w_ref, g1b_ref, g2w_ref, g2b_ref,
                l1w_ref, l1b_ref, l2w_ref, l2b_ref, o_ref):
    for e in range(2):
        g = gs_ref[e:e + 1, :] * (1.0 / N)
        t = _elu(_mm(g, g1w_ref[e]) + g1b_ref[e:e + 1, :])
        ge = _mm(t, g2w_ref[e]) + g2b_ref[e:e + 1, :]
        t = _elu(_mm(ge, l1w_ref[e]) + l1b_ref[e:e + 1, :])
        y = _mm(t, l2w_ref[e]) + l2b_ref[e:e + 1, :]
        nrm = jnp.sqrt(jnp.sum(y * y, axis=-1, keepdims=True))
        o_ref[e:e + 1, :] = y / jnp.maximum(nrm, 1e-12)


_graph_call = pl.pallas_call(
    _graph_body,
    out_shape=jax.ShapeDtypeStruct((2, PROJ), jnp.float32),
)


def kernel(sc_x, fc_x, params, sc_edge_index, fc_edge_index):
    z_node = {}
    gsum = {}
    for name, x, ei in (("sc", sc_x, sc_edge_index), ("fc", fc_x, fc_edge_index)):
        enc = params[name + "_enc"]
        src_p, dst_p = _pad_edges(ei)
        h = _linear(x, enc["input_proj"])
        for lp in enc["layers"]:
            agg2 = _segsum(h, src_p, dst_p)
            h = _gin(h, agg2, lp)
        z_node[name], gsum[name] = _node_head(
            h, enc["node_proj"], params[name + "_node_proj"])

    def stk(fn):
        return jnp.stack([fn("sc"), fn("fc")])

    gs = jnp.concatenate([gsum["sc"], gsum["fc"]], axis=0)
    zg = _graph_call(
        gs,
        stk(lambda n: params[n + "_enc"]["graph_proj1"]["W"]),
        stk(lambda n: params[n + "_enc"]["graph_proj1"]["b"].reshape(1, D)[0]),
        stk(lambda n: params[n + "_enc"]["graph_proj2"]["W"]),
        stk(lambda n: params[n + "_enc"]["graph_proj2"]["b"].reshape(1, D)[0]),
        stk(lambda n: params[n + "_proj"]["l1"]["W"]),
        stk(lambda n: params[n + "_proj"]["l1"]["b"]),
        stk(lambda n: params[n + "_proj"]["l2"]["W"]),
        stk(lambda n: params[n + "_proj"]["l2"]["b"]),
    )
    return (zg[0:1], zg[1:2], z_node["sc"], z_node["fc"])
